# baseline (speedup 1.0000x reference)
import functools

import jax
import jax.numpy as jnp
from jax import lax
from jax.experimental import pallas as pl
from jax.experimental.pallas import tpu as pltpu
from jax.experimental.pallas import tpu_sc as plsc

D = 64
B = 16384
J = 50
NC, NS = 2, 16
NW = NC * NS
CH = 128
IB_PER_W = (B // CH) // NW
WCOLS = IB_PER_W * CH
NBUF = 4
NTR = 2

_mesh = plsc.VectorSubcoreMesh(core_axis_name="c", subcore_axis_name="s")

V = 1000000
TCOLS = 7813
Q_PAD = TCOLS * 64
NPB = 6
PT_MAX = -(-TCOLS // NW)


@functools.partial(
    pl.kernel,
    mesh=_mesh,
    out_type=jax.ShapeDtypeStruct((Q_PAD, 128), jnp.float32),
    compiler_params=pltpu.CompilerParams(
        use_tc_tiling_on_sc=True, needs_layout_passes=False),
    scratch_types=[
        pltpu.VMEM((NPB, D, 128), jnp.float32),
        pltpu.VMEM((NPB, D, 128), jnp.float32),
    ] + [pltpu.SemaphoreType.DMA] * (2 * NPB),
)
def _pack_kernel(tT_hbm, t2_hbm, in_v, out_v, *sems):
    isems = sems[:NPB]
    osems = sems[NPB:]
    wid = lax.axis_index("s") * NC + lax.axis_index("c")
    lane = jnp.arange(16, dtype=jnp.int32)
    lane2 = lane * 2
    diag = [(lane + d0) & 15 for d0 in range(16)]
    qvecs = [lane + 16 * g for g in range(4)]

    def tc_of(t):
        return wid + NW * t

    def start_in(pb, t):
        pltpu.make_async_copy(
            tT_hbm.at[:, pl.ds(tc_of(t) * 128, 128)], in_v.at[pb],
            isems[pb]).start()

    def wait_in(pb, t):
        pltpu.make_async_copy(
            tT_hbm.at[:, pl.ds(tc_of(t) * 128, 128)], in_v.at[pb],
            isems[pb]).wait()

    def start_out(pb, t):
        pltpu.make_async_copy(
            out_v.at[pb], t2_hbm.at[pl.ds(tc_of(t) * 64, 64)],
            osems[pb]).start()

    def wait_out(pb, t):
        pltpu.make_async_copy(
            out_v.at[pb], t2_hbm.at[pl.ds(tc_of(t) * 64, 64)],
            osems[pb]).wait()

    for pb in range(NPB):
        @pl.when(tc_of(pb) < TCOLS)
        def _():
            start_in(pb, pb)

    def body(tg, carry):
        for pb in range(NPB):
            t = tg * NPB + pb
            _pack_one(pb, t)
        return carry

    def _pack_one(pb, t):
        @pl.when(tc_of(t) < TCOLS)
        def _():
            wait_in(pb, t)

            @pl.when(t >= NPB)
            def _():
                wait_out(pb, t - NPB)

            def pkstep(pk, c):
                p = pk >> 2
                k = pk & 3
                dk = [diag[d0] + k * 16 for d0 in range(16)]
                ck = [dk[d0] + p * 64 for d0 in range(16)]
                for g in range(4):
                    clv = lane2 + g * 32 + p
                    vals = [
                        plsc.load_gather(in_v.at[pb], [dk[d0], clv])
                        for d0 in range(16)
                    ]
                    for d0 in range(16):
                        plsc.store_scatter(
                            out_v.at[pb], [qvecs[g], ck[d0]], vals[d0])
                return c

            lax.fori_loop(0, 8, pkstep, 0)

            start_out(pb, t)

            @pl.when(tc_of(t + NPB) < TCOLS)
            def _():
                start_in(pb, t + NPB)

    ngroups = -(-PT_MAX // NPB)
    lax.fori_loop(0, ngroups, body, 0)
    for pb in range(NPB):
        wait_out(pb, pb)


@functools.partial(
    pl.kernel,
    mesh=_mesh,
    out_type=jax.ShapeDtypeStruct((J, D, B), jnp.float32),
    compiler_params=pltpu.CompilerParams(
        use_tc_tiling_on_sc=True, needs_layout_passes=False),
    scratch_types=[
        pltpu.VMEM((J, WCOLS), jnp.int32),
        pltpu.VMEM((NBUF, CH), jnp.int32),
        pltpu.VMEM((NBUF, CH, 128), jnp.float32),
        pltpu.VMEM((NTR, D, CH), jnp.float32),
    ] + [pltpu.SemaphoreType.DMA] * (NBUF + NTR),
)
def _embed_kernel(table2_hbm, xt_hbm, out_hbm, idx_all, q_v, rows_v, tr_v,
                  *sems):
    gsems = sems[:NBUF]
    ssems = sems[NBUF:]
    wid = lax.axis_index("s") * NC + lax.axis_index("c")
    lane = jnp.arange(16, dtype=jnp.int32)
    rowvecs = [lane + (g * 16) for g in range(CH // 16)]
    diag = [(lane + d0) & 15 for d0 in range(16)]

    def i0_of(b):
        return (wid * IB_PER_W + b) * CH

    def qcompute(b, j):
        for k in range(CH // 16):
            v = idx_all[j, pl.ds(b * CH + k * 16, 16)]
            q_v[b, pl.ds(k * 16, 16)] = lax.shift_right_logical(v, 1)

    def start_gather(b):
        pltpu.make_async_copy(
            table2_hbm.at[q_v.at[b]], rows_v.at[b], gsems[b]).start()

    def wait_gather(b):
        pltpu.make_async_copy(
            table2_hbm.at[q_v.at[b]], rows_v.at[b], gsems[b]).wait()

    def start_store(tb, j, b):
        pltpu.make_async_copy(
            tr_v.at[tb], out_hbm.at[j, :, pl.ds(i0_of(b), CH)],
            ssems[tb]).start()

    def wait_store(tb):
        pltpu.make_async_copy(
            tr_v.at[tb], out_hbm.at[0, :, pl.ds(i0_of(0), CH)],
            ssems[tb]).wait()

    pltpu.sync_copy(xt_hbm.at[:, pl.ds(wid * WCOLS, WCOLS)], idx_all)

    for b in range(NBUF):
        qcompute(b, 0)
        start_gather(b)

    def body(j, carry):
        for b in range(NBUF):
            tb = b % NTR
            wait_gather(b)

            if b >= NTR:
                wait_store(tb)
            else:
                @pl.when(j > 0)
                def _():
                    wait_store(tb)

            pvec = [
                (idx_all[j, pl.ds(b * CH + g * 16, 16)] & 1) * D
                for g in range(CH // 16)
            ]

            def kstep(kk, c):
                koff = kk * 16
                dk = [diag[d0] + koff for d0 in range(16)]
                for g in range(CH // 16):
                    colk = pvec[g] + koff
                    vals = [
                        plsc.load_gather(
                            rows_v.at[b], [rowvecs[g], colk + diag[d0]])
                        for d0 in range(16)
                    ]
                    for d0 in range(16):
                        plsc.store_scatter(
                            tr_v.at[tb], [dk[d0], rowvecs[g]], vals[d0])
                return c

            lax.fori_loop(0, D // 16, kstep, 0)
            start_store(tb, j, b)

            @pl.when(j < J - 1)
            def _():
                qcompute(b, j + 1)
                start_gather(b)
        return carry

    lax.fori_loop(0, J, body, 0)
    for tb in range(NTR):
        wait_store(tb)


def kernel(x, table):
    table2 = _pack_kernel(table.T)
    xt = x.T.astype(jnp.int32)
    out_t = _embed_kernel(table2, xt)
    return out_t.transpose(2, 0, 1)


# speedup vs baseline: 1.0039x; 1.0039x over previous
import functools

import jax
import jax.numpy as jnp
from jax import lax
from jax.experimental import pallas as pl
from jax.experimental.pallas import tpu as pltpu
from jax.experimental.pallas import tpu_sc as plsc

D = 64
B = 16384
J = 50
NC, NS = 2, 16
NW = NC * NS
CH = 128
IB_PER_W = (B // CH) // NW
WCOLS = IB_PER_W * CH
NBUF = 4
NTR = 2

_mesh = plsc.VectorSubcoreMesh(core_axis_name="c", subcore_axis_name="s")

V = 1000000
TCOLS = 7813
Q_PAD = TCOLS * 64
NPB = 4
PT_MAX = -(-TCOLS // NW)


@functools.partial(
    pl.kernel,
    mesh=_mesh,
    out_type=jax.ShapeDtypeStruct((Q_PAD, 128), jnp.float32),
    compiler_params=pltpu.CompilerParams(
        use_tc_tiling_on_sc=True, needs_layout_passes=False),
    scratch_types=[
        pltpu.VMEM((NPB, D, 128), jnp.float32),
        pltpu.VMEM((NPB, D, 128), jnp.float32),
    ] + [pltpu.SemaphoreType.DMA] * (2 * NPB),
)
def _pack_kernel(tT_hbm, t2_hbm, in_v, out_v, *sems):
    isems = sems[:NPB]
    osems = sems[NPB:]
    wid = lax.axis_index("s") * NC + lax.axis_index("c")
    lane = jnp.arange(16, dtype=jnp.int32)
    lane2 = lane * 2
    diag = [(lane + d0) & 15 for d0 in range(16)]
    qvecs = [lane + 16 * g for g in range(4)]

    def tc_of(t):
        return wid + NW * t

    def start_in(pb, t):
        pltpu.make_async_copy(
            tT_hbm.at[:, pl.ds(tc_of(t) * 128, 128)], in_v.at[pb],
            isems[pb]).start()

    def wait_in(pb, t):
        pltpu.make_async_copy(
            tT_hbm.at[:, pl.ds(tc_of(t) * 128, 128)], in_v.at[pb],
            isems[pb]).wait()

    def start_out(pb, t):
        pltpu.make_async_copy(
            out_v.at[pb], t2_hbm.at[pl.ds(tc_of(t) * 64, 64)],
            osems[pb]).start()

    def wait_out(pb, t):
        pltpu.make_async_copy(
            out_v.at[pb], t2_hbm.at[pl.ds(tc_of(t) * 64, 64)],
            osems[pb]).wait()

    for pb in range(NPB):
        @pl.when(tc_of(pb) < TCOLS)
        def _():
            start_in(pb, pb)

    def body(tg, carry):
        for pb in range(NPB):
            t = tg * NPB + pb
            _pack_one(pb, t)
        return carry

    def _pack_one(pb, t):
        @pl.when(tc_of(t) < TCOLS)
        def _():
            wait_in(pb, t)

            @pl.when(t >= NPB)
            def _():
                wait_out(pb, t - NPB)

            def pkstep(pk, c):
                p = pk >> 2
                k = pk & 3
                dk = [diag[d0] + k * 16 for d0 in range(16)]
                ck = [dk[d0] + p * 64 for d0 in range(16)]
                for g in range(4):
                    clv = lane2 + g * 32 + p
                    vals = [
                        plsc.load_gather(in_v.at[pb], [dk[d0], clv])
                        for d0 in range(16)
                    ]
                    for d0 in range(16):
                        plsc.store_scatter(
                            out_v.at[pb], [qvecs[g], ck[d0]], vals[d0])
                return c

            lax.fori_loop(0, 8, pkstep, 0)

            start_out(pb, t)

            @pl.when(tc_of(t + NPB) < TCOLS)
            def _():
                start_in(pb, t + NPB)

    ngroups = -(-PT_MAX // NPB)
    lax.fori_loop(0, ngroups, body, 0)
    for pb in range(NPB):
        wait_out(pb, pb)


@functools.partial(
    pl.kernel,
    mesh=_mesh,
    out_type=jax.ShapeDtypeStruct((J, D, B), jnp.float32),
    compiler_params=pltpu.CompilerParams(
        use_tc_tiling_on_sc=True, needs_layout_passes=False),
    scratch_types=[
        pltpu.VMEM((J, WCOLS), jnp.int32),
        pltpu.VMEM((NBUF, CH), jnp.int32),
        pltpu.VMEM((NBUF, CH, 128), jnp.float32),
        pltpu.VMEM((NTR, D, CH), jnp.float32),
    ] + [pltpu.SemaphoreType.DMA] * (NBUF + NTR),
)
def _embed_kernel(table2_hbm, xt_hbm, out_hbm, idx_all, q_v, rows_v, tr_v,
                  *sems):
    gsems = sems[:NBUF]
    ssems = sems[NBUF:]
    wid = lax.axis_index("s") * NC + lax.axis_index("c")
    lane = jnp.arange(16, dtype=jnp.int32)
    rowvecs = [lane + (g * 16) for g in range(CH // 16)]
    diag = [(lane + d0) & 15 for d0 in range(16)]

    def i0_of(b):
        return (wid * IB_PER_W + b) * CH

    def qcompute(b, j):
        for k in range(CH // 16):
            v = idx_all[j, pl.ds(b * CH + k * 16, 16)]
            q_v[b, pl.ds(k * 16, 16)] = lax.shift_right_logical(v, 1)

    def start_gather(b):
        pltpu.make_async_copy(
            table2_hbm.at[q_v.at[b]], rows_v.at[b], gsems[b]).start()

    def wait_gather(b):
        pltpu.make_async_copy(
            table2_hbm.at[q_v.at[b]], rows_v.at[b], gsems[b]).wait()

    def start_store(tb, j, b):
        pltpu.make_async_copy(
            tr_v.at[tb], out_hbm.at[j, :, pl.ds(i0_of(b), CH)],
            ssems[tb]).start()

    def wait_store(tb):
        pltpu.make_async_copy(
            tr_v.at[tb], out_hbm.at[0, :, pl.ds(i0_of(0), CH)],
            ssems[tb]).wait()

    pltpu.sync_copy(xt_hbm.at[:, pl.ds(wid * WCOLS, WCOLS)], idx_all)

    for b in range(NBUF):
        qcompute(b, 0)
        start_gather(b)

    def body(j, carry):
        for b in range(NBUF):
            tb = b % NTR
            wait_gather(b)

            if b >= NTR:
                wait_store(tb)
            else:
                @pl.when(j > 0)
                def _():
                    wait_store(tb)

            pvec = [
                (idx_all[j, pl.ds(b * CH + g * 16, 16)] & 1) * D
                for g in range(CH // 16)
            ]

            def kstep(kk, c):
                koff = kk * 16
                dk = [diag[d0] + koff for d0 in range(16)]
                for g in range(CH // 16):
                    colk = pvec[g] + koff
                    vals = [
                        plsc.load_gather(
                            rows_v.at[b], [rowvecs[g], colk + diag[d0]])
                        for d0 in range(16)
                    ]
                    for d0 in range(16):
                        plsc.store_scatter(
                            tr_v.at[tb], [dk[d0], rowvecs[g]], vals[d0])
                return c

            lax.fori_loop(0, D // 16, kstep, 0)
            start_store(tb, j, b)

            @pl.when(j < J - 1)
            def _():
                qcompute(b, j + 1)
                start_gather(b)
        return carry

    lax.fori_loop(0, J, body, 0)
    for tb in range(NTR):
        wait_store(tb)


def kernel(x, table):
    table2 = _pack_kernel(table.T)
    xt = x.T.astype(jnp.int32)
    out_t = _embed_kernel(table2, xt)
    return out_t.transpose(2, 0, 1)
